# Initial kernel scaffold; baseline (speedup 1.0000x reference)
import jax
import jax.numpy as jnp
from jax.experimental import pallas as pl


def kernel(params, x_idx, intra_ei, intra_ea_idx, global_ei, global_ea_idx, node_ids, batch_assign, valid):
    raise NotImplementedError("write your pallas kernel here")


# speedup vs baseline: 1.0573x; 1.0573x over previous
import functools
import jax
import jax.numpy as jnp
from jax.experimental import pallas as pl
from jax.experimental.pallas import tpu as pltpu

_N_TOTAL = 1024
_M_SUB = 2
_K = 32
_S = _N_TOTAL * _M_SUB
_SK = _S * _K
_H = 128
_T_RWSE = 16
_N_GRAPHS = 64

_ROW_BLOCK = 1024


def _mlp_ln_body(hl_ref, hb_ref, hprev_ref, w1a_ref, w1b_ref, b1_ref,
                 w2_ref, b2_ref, lng_ref, lnb_ref, out_ref):
    hl = hl_ref[...]
    hb = hb_ref[...]
    z = jnp.dot(hl, w1a_ref[...], preferred_element_type=jnp.float32)
    z += jnp.dot(hb, w1b_ref[...], preferred_element_type=jnp.float32)
    z += b1_ref[...]
    z = jax.nn.gelu(z)
    z = jnp.dot(z, w2_ref[...], preferred_element_type=jnp.float32) + b2_ref[...]
    mu = jnp.mean(z, axis=-1, keepdims=True)
    var = jnp.mean((z - mu) ** 2, axis=-1, keepdims=True)
    z = (z - mu) * jax.lax.rsqrt(var + 1e-5) * lng_ref[...] + lnb_ref[...]
    out_ref[...] = hprev_ref[...] + z


@functools.partial(jax.jit, static_argnames=())
def _fused_mlp_ln(h_local, h_bcast, h_prev, w1a, w1b, b1, w2, b2, lng, lnb):
    n = h_local.shape[0]
    grid = (n // _ROW_BLOCK,)
    row_spec = pl.BlockSpec((_ROW_BLOCK, _H), lambda i: (i, 0))
    full = pl.BlockSpec((_H, _H), lambda i: (0, 0))
    vec = pl.BlockSpec((_H,), lambda i: (0,))
    return pl.pallas_call(
        _mlp_ln_body,
        grid=grid,
        in_specs=[row_spec, row_spec, row_spec, full, full, vec, full, vec,
                  vec, vec],
        out_specs=row_spec,
        out_shape=jax.ShapeDtypeStruct((n, _H), jnp.float32),
    )(h_local, h_bcast, h_prev, w1a, w1b, b1, w2, b2, lng, lnb)


def _bn(x, g, b):
    mu = x.mean(0)
    var = x.var(0)
    return (x - mu) / jnp.sqrt(var + 1e-5) * g + b


def _ln(x, g, b):
    mu = x.mean(-1, keepdims=True)
    var = x.var(-1, keepdims=True)
    return (x - mu) / jnp.sqrt(var + 1e-5) * g + b


def _gine(h, ei, ea, eps, W1, b1, W2, b2, n):
    msg = jax.nn.relu(h[ei[0]] + ea)
    agg = jax.ops.segment_sum(msg, ei[1], num_segments=n)
    z = (1.0 + eps) * h + agg
    return jax.nn.relu(z @ W1 + b1) @ W2 + b2


def _rwse(global_ei, n, t):
    A = jnp.zeros((n, n), dtype=jnp.float32).at[global_ei[0], global_ei[1]].add(1.0)
    deg = jnp.clip(A.sum(1), 1.0, None)
    P = A / deg[:, None]
    half = t // 2
    pows = [P]
    for _ in range(half - 1):
        pows.append(pows[-1] @ P)
    diags = []
    for step in range(1, t + 1):
        a = (step + 1) // 2
        b = step // 2
        if b == 0:
            diags.append(jnp.diagonal(pows[a - 1]))
        else:
            diags.append(jnp.sum(pows[a - 1] * pows[b - 1].T, axis=1))
    return jnp.stack(diags, axis=1)


def kernel(params, x_idx, intra_ei, intra_ea_idx, global_ei, global_ea_idx,
           node_ids, batch_assign, valid):
    rwse = _rwse(global_ei, _N_TOTAL, _T_RWSE)
    h0 = params["atom_emb"][x_idx]
    rwse_flat = jax.nn.relu(rwse @ params["rwse_W"] + params["rwse_b"])[node_ids]
    valid_f = valid.astype(jnp.float32)[:, None]
    h = (h0 + rwse_flat) * valid_f
    ea_local = params["bond_emb"][intra_ea_idx]
    ea_global = params["bond_emb"][global_ea_idx]
    root_flat_idx = jnp.arange(_S) * _K
    node_assign = jnp.repeat(jnp.arange(_N_TOTAL), _M_SUB)
    sub_ids = jnp.arange(_SK) // _K
    for lp in params["layers"]:
        h_local = _bn(jax.nn.relu(_gine(h, intra_ei, ea_local, lp["eps_l"],
                                        lp["W1_l"], lp["b1_l"], lp["W2_l"],
                                        lp["b2_l"], _SK)),
                      lp["bn_g_l"], lp["bn_b_l"])
        h_root = h[root_flat_idx]
        h_node = jax.ops.segment_sum(h_root, node_assign,
                                     num_segments=_N_TOTAL) / float(_M_SUB)
        h_cross = _bn(jax.nn.relu(_gine(h_node, global_ei, ea_global,
                                        lp["eps_g"], lp["W1_g"], lp["b1_g"],
                                        lp["W2_g"], lp["b2_g"], _N_TOTAL)),
                      lp["bn_g_g"], lp["bn_b_g"])
        h_node = h_node + h_cross
        h_bcast = (h_node[node_assign] @ lp["W_bcast"])[sub_ids]
        w1a = lp["cW1"][:_H]
        w1b = lp["cW1"][_H:]
        h_new = _fused_mlp_ln(h_local, h_bcast, h, w1a, w1b, lp["cb1"],
                              lp["cW2"], lp["cb2"], lp["ln_g"], lp["ln_b"])
        h = h_new * valid_f
    hv = h * valid_f
    cnt = jax.ops.segment_sum(valid_f[:, 0], sub_ids, num_segments=_S)
    h_sub = jax.ops.segment_sum(hv, sub_ids, num_segments=_S) / jnp.clip(cnt, 1.0, None)[:, None]
    h_node_f = h_sub.reshape(_N_TOTAL, _M_SUB, _H).mean(axis=1)
    return jax.ops.segment_sum(h_node_f, batch_assign, num_segments=_N_GRAPHS)


# speedup vs baseline: 1.3395x; 1.2669x over previous
import functools
import jax
import jax.numpy as jnp
from jax import lax
from jax.experimental import pallas as pl
from jax.experimental.pallas import tpu as pltpu
from jax.experimental.pallas import tpu_sc as plsc

_N_TOTAL = 1024
_M_SUB = 2
_K = 32
_S = _N_TOTAL * _M_SUB
_SK = _S * _K
_H = 128
_T_RWSE = 16
_N_GRAPHS = 64
_EDGE_DIM = 16

_ROW_BLOCK = 1024
_CH = 128
_LANES = 16
_N_TILES = 16
_N_CORES = 2


def _make_edge_agg(n_edges, n_rows, win_rows, n_win):
    epc = n_edges // _N_TILES
    n_chunks = epc // _CH
    pre = min(epc, 2048)
    n_pre = epc // pre
    rpt = win_rows // _N_TILES
    mesh = plsc.VectorSubcoreMesh(core_axis_name="c", subcore_axis_name="s",
                                  num_cores=_N_CORES, num_subcores=_N_TILES)

    @functools.partial(
        pl.kernel, mesh=mesh,
        compiler_params=pltpu.CompilerParams(needs_layout_passes=False),
        out_type=jax.ShapeDtypeStruct((n_rows, _H), jnp.float32),
        scratch_types=[
            pltpu.VMEM((_CH, _H), jnp.float32),
            pltpu.VMEM((n_chunks, _CH), jnp.int32),
            pltpu.VMEM((n_chunks, _CH), jnp.int32),
            pltpu.VMEM((n_chunks, _CH), jnp.int32),
            pltpu.VMEM((pre,), jnp.int32),
            pltpu.VMEM((pre,), jnp.int32),
            pltpu.VMEM((pre,), jnp.int32),
            pltpu.VMEM((_EDGE_DIM, _H), jnp.float32),
            pltpu.VMEM_SHARED((win_rows, _H), jnp.float32),
        ],
    )
    def k(h_hbm, src_hbm, dst_hbm, et_hbm, bond_hbm, out_hbm,
          msgbuf, slist, dlist, tlist, s_st, d_st, t_st, bond_v, acc):
        c = lax.axis_index("c")
        s = lax.axis_index("s")
        ebase = s * epc
        zf = jnp.zeros((_LANES,), jnp.float32)
        zi = jnp.zeros((_LANES,), jnp.int32)
        pltpu.sync_copy(bond_hbm, bond_v)

        for w in range(n_win):
            win_lo = (c * n_win + w) * win_rows

            def zrow(r, carry):
                for j in range(8):
                    msgbuf[r, pl.ds(j * _LANES, _LANES)] = zf
                return carry
            lax.fori_loop(0, _CH, zrow, 0)

            for q in range(max(1, rpt // _CH)):
                rows = min(rpt, _CH)
                pltpu.sync_copy(msgbuf.at[pl.ds(0, rows)],
                                acc.at[pl.ds(s * rpt + q * _CH, rows)])

            def zlist(i, carry):
                for j in range(8):
                    sl = pl.ds(j * _LANES, _LANES)
                    slist[i, sl] = zi
                    dlist[i, sl] = zi
                    tlist[i, sl] = zi
                return carry
            lax.fori_loop(0, n_chunks, zlist, 0)

            def prep_stage(p, cnt):
                start = ebase + p * pre
                pltpu.sync_copy(src_hbm.at[pl.ds(start, pre)], s_st)
                pltpu.sync_copy(dst_hbm.at[pl.ds(start, pre)], d_st)
                pltpu.sync_copy(et_hbm.at[pl.ds(start, pre)], t_st)

                def vec_step(i, cnt):
                    sl = pl.ds(i * _LANES, _LANES)
                    d = d_st[sl]
                    msk = (d >= win_lo) & (d < win_lo + win_rows)
                    pos = cnt + plsc.cumsum(jnp.where(msk, 1, 0)) - 1
                    chi = lax.shift_right_arithmetic(pos, 7)
                    off = lax.bitwise_and(pos, _CH - 1)
                    plsc.store_scatter(slist, [chi, off], s_st[sl], mask=msk)
                    plsc.store_scatter(dlist, [chi, off], d - win_lo, mask=msk)
                    plsc.store_scatter(tlist, [chi, off], t_st[sl], mask=msk)
                    return jnp.max(pos) + 1
                return lax.fori_loop(0, pre // _LANES, vec_step, cnt)
            cnt = lax.fori_loop(0, n_pre, prep_stage, jnp.int32(0))

            plsc.subcore_barrier()

            def chunk_body(ch, carry):
                @pl.when(ch * _CH < cnt)
                def _():
                    pltpu.sync_copy(h_hbm.at[slist.at[ch]], msgbuf)

                    def relu_group(g, carry2):
                        et_vec = tlist[ch, pl.ds(g * _LANES, _LANES)]
                        for rr in range(_LANES):
                            r = g * _LANES + rr
                            et = et_vec[rr]
                            live = (ch * _CH + r) < cnt
                            mf = jnp.where(live, 1.0, 0.0)
                            for j in range(8):
                                sl = pl.ds(j * _LANES, _LANES)
                                v = msgbuf[r, sl] + bond_v[et, sl]
                                msgbuf[r, sl] = jnp.maximum(v, 0.0) * mf
                        return carry2
                    lax.fori_loop(0, _CH // _LANES, relu_group, 0)

                    pltpu.sync_copy(msgbuf, acc.at[dlist.at[ch]], add=True)
                return carry
            lax.fori_loop(0, n_chunks, chunk_body, 0)

            plsc.subcore_barrier()

            pltpu.sync_copy(acc.at[pl.ds(s * rpt, rpt)],
                            out_hbm.at[pl.ds(win_lo + s * rpt, rpt)])
            plsc.subcore_barrier()

    return k


_edge_agg_intra = _make_edge_agg(262144, _SK, 4096, 8)
_edge_agg_global = _make_edge_agg(16384, _N_TOTAL, 512, 1)


def _mlp_ln_body(hl_ref, hb_ref, hprev_ref, w1a_ref, w1b_ref, b1_ref,
                 w2_ref, b2_ref, lng_ref, lnb_ref, out_ref):
    hl = hl_ref[...]
    hb = hb_ref[...]
    z = jnp.dot(hl, w1a_ref[...], preferred_element_type=jnp.float32)
    z += jnp.dot(hb, w1b_ref[...], preferred_element_type=jnp.float32)
    z += b1_ref[...]
    z = jax.nn.gelu(z)
    z = jnp.dot(z, w2_ref[...], preferred_element_type=jnp.float32) + b2_ref[...]
    mu = jnp.mean(z, axis=-1, keepdims=True)
    var = jnp.mean((z - mu) ** 2, axis=-1, keepdims=True)
    z = (z - mu) * jax.lax.rsqrt(var + 1e-5) * lng_ref[...] + lnb_ref[...]
    out_ref[...] = hprev_ref[...] + z


def _fused_mlp_ln(h_local, h_bcast, h_prev, w1a, w1b, b1, w2, b2, lng, lnb):
    n = h_local.shape[0]
    grid = (n // _ROW_BLOCK,)
    row_spec = pl.BlockSpec((_ROW_BLOCK, _H), lambda i: (i, 0))
    full = pl.BlockSpec((_H, _H), lambda i: (0, 0))
    vec = pl.BlockSpec((_H,), lambda i: (0,))
    return pl.pallas_call(
        _mlp_ln_body,
        grid=grid,
        in_specs=[row_spec, row_spec, row_spec, full, full, vec, full, vec,
                  vec, vec],
        out_specs=row_spec,
        out_shape=jax.ShapeDtypeStruct((n, _H), jnp.float32),
    )(h_local, h_bcast, h_prev, w1a, w1b, b1, w2, b2, lng, lnb)


def _bn(x, g, b):
    mu = x.mean(0)
    var = x.var(0)
    return (x - mu) / jnp.sqrt(var + 1e-5) * g + b


def _rwse(global_ei, n, t):
    A = jnp.zeros((n, n), dtype=jnp.float32).at[global_ei[0], global_ei[1]].add(1.0)
    deg = jnp.clip(A.sum(1), 1.0, None)
    P = A / deg[:, None]
    half = t // 2
    pows = [P]
    for _ in range(half - 1):
        pows.append(pows[-1] @ P)
    diags = []
    for step in range(1, t + 1):
        a = (step + 1) // 2
        b = step // 2
        if b == 0:
            diags.append(jnp.diagonal(pows[a - 1]))
        else:
            diags.append(jnp.sum(pows[a - 1] * pows[b - 1].T, axis=1))
    return jnp.stack(diags, axis=1)


def kernel(params, x_idx, intra_ei, intra_ea_idx, global_ei, global_ea_idx,
           node_ids, batch_assign, valid):
    bond = params["bond_emb"]
    rwse = _rwse(global_ei, _N_TOTAL, _T_RWSE)
    h0 = params["atom_emb"][x_idx]
    rwse_flat = jax.nn.relu(rwse @ params["rwse_W"] + params["rwse_b"])[node_ids]
    valid_f = valid.astype(jnp.float32)[:, None]
    h = (h0 + rwse_flat) * valid_f
    isrc = intra_ei[0]
    idst = intra_ei[1]
    gsrc = global_ei[0]
    gdst = global_ei[1]
    for lp in params["layers"]:
        agg = _edge_agg_intra(h, isrc, idst, intra_ea_idx, bond)
        z = (1.0 + lp["eps_l"]) * h + agg
        u = jax.nn.relu(jax.nn.relu(z @ lp["W1_l"] + lp["b1_l"]) @ lp["W2_l"]
                        + lp["b2_l"])
        h_local = _bn(u, lp["bn_g_l"], lp["bn_b_l"])

        h_node = h.reshape(_N_TOTAL, _M_SUB * _K, _H)[:, ::_K].mean(axis=1)
        agg_g = _edge_agg_global(h_node, gsrc, gdst, global_ea_idx, bond)
        zg = (1.0 + lp["eps_g"]) * h_node + agg_g
        ug = jax.nn.relu(jax.nn.relu(zg @ lp["W1_g"] + lp["b1_g"]) @ lp["W2_g"]
                         + lp["b2_g"])
        h_node = h_node + _bn(ug, lp["bn_g_g"], lp["bn_b_g"])

        hb_node = h_node @ lp["W_bcast"]
        h_bcast = jnp.repeat(hb_node, _M_SUB * _K, axis=0)

        w1a = lp["cW1"][:_H]
        w1b = lp["cW1"][_H:]
        h_new = _fused_mlp_ln(h_local, h_bcast, h, w1a, w1b, lp["cb1"],
                              lp["cW2"], lp["cb2"], lp["ln_g"], lp["ln_b"])
        h = h_new * valid_f
    hv = h * valid_f
    cnt = valid_f[:, 0].reshape(_S, _K).sum(axis=1)
    h_sub = hv.reshape(_S, _K, _H).sum(axis=1) / jnp.clip(cnt, 1.0, None)[:, None]
    h_node_f = h_sub.reshape(_N_TOTAL, _M_SUB, _H).mean(axis=1)
    onehot = (batch_assign[None, :] == jnp.arange(_N_GRAPHS)[:, None]
              ).astype(jnp.float32)
    return onehot @ h_node_f


# speedup vs baseline: 1.4384x; 1.0738x over previous
import functools
import jax
import jax.numpy as jnp
from jax import lax
from jax.experimental import pallas as pl
from jax.experimental.pallas import tpu as pltpu
from jax.experimental.pallas import tpu_sc as plsc

_N_TOTAL = 1024
_M_SUB = 2
_K = 32
_S = _N_TOTAL * _M_SUB
_SK = _S * _K
_H = 128
_T_RWSE = 16
_N_GRAPHS = 64
_EDGE_DIM = 16

_ROW_BLOCK = 1024
_CH = 128
_LANES = 16
_N_TILES = 16
_N_CORES = 2


def _make_edge_agg(n_edges, n_rows, win_rows, n_win):
    epc = n_edges // _N_TILES
    n_chunks = epc // _CH
    pre = min(epc, 2048)
    n_pre = epc // pre
    rpt = win_rows // _N_TILES
    mesh = plsc.VectorSubcoreMesh(core_axis_name="c", subcore_axis_name="s",
                                  num_cores=_N_CORES, num_subcores=_N_TILES)

    @functools.partial(
        pl.kernel, mesh=mesh,
        compiler_params=pltpu.CompilerParams(needs_layout_passes=False),
        out_type=jax.ShapeDtypeStruct((n_rows, _H), jnp.float32),
        scratch_types=[
            pltpu.VMEM((_CH, _H), jnp.float32),
            pltpu.VMEM((_CH, _H), jnp.float32),
            pltpu.VMEM((n_chunks, _CH), jnp.int32),
            pltpu.VMEM((n_chunks, _CH), jnp.int32),
            pltpu.VMEM((n_chunks, _CH), jnp.int32),
            pltpu.VMEM((pre,), jnp.int32),
            pltpu.VMEM((pre,), jnp.int32),
            pltpu.VMEM((pre,), jnp.int32),
            pltpu.VMEM((_EDGE_DIM, _H), jnp.float32),
            pltpu.VMEM_SHARED((win_rows, _H), jnp.float32),
            pltpu.SemaphoreType.DMA,
            pltpu.SemaphoreType.DMA,
            pltpu.SemaphoreType.DMA,
            pltpu.SemaphoreType.DMA,
        ],
    )
    def k(h_hbm, src_hbm, dst_hbm, et_hbm, bond_hbm, out_hbm,
          msgbuf, msgbuf2, slist, dlist, tlist, s_st, d_st, t_st, bond_v, acc,
          gsa, gsb, ssa, ssb):
        c = lax.axis_index("c")
        s = lax.axis_index("s")
        ebase = s * epc
        zf = jnp.zeros((_LANES,), jnp.float32)
        zi = jnp.zeros((_LANES,), jnp.int32)
        pltpu.sync_copy(bond_hbm, bond_v)

        def win_body(w, wcarry):
            win_lo = (c * n_win + w) * win_rows

            def zrow(r, carry):
                for j in range(8):
                    msgbuf[r, pl.ds(j * _LANES, _LANES)] = zf
                return carry
            lax.fori_loop(0, _CH, zrow, 0)

            for q in range(max(1, rpt // _CH)):
                rows = min(rpt, _CH)
                pltpu.sync_copy(msgbuf.at[pl.ds(0, rows)],
                                acc.at[pl.ds(s * rpt + q * _CH, rows)])

            def zlist(i, carry):
                for j in range(8):
                    sl = pl.ds(j * _LANES, _LANES)
                    slist[i, sl] = zi
                    dlist[i, sl] = zi
                    tlist[i, sl] = zi
                return carry
            lax.fori_loop(0, n_chunks, zlist, 0)

            def prep_stage(p, cnt):
                start = ebase + p * pre
                pltpu.sync_copy(src_hbm.at[pl.ds(start, pre)], s_st)
                pltpu.sync_copy(dst_hbm.at[pl.ds(start, pre)], d_st)
                pltpu.sync_copy(et_hbm.at[pl.ds(start, pre)], t_st)

                def vec_step(i, cnt):
                    sl = pl.ds(i * _LANES, _LANES)
                    d = d_st[sl]
                    msk = (d >= win_lo) & (d < win_lo + win_rows)
                    pos = cnt + plsc.cumsum(jnp.where(msk, 1, 0)) - 1
                    chi = lax.shift_right_arithmetic(pos, 7)
                    off = lax.bitwise_and(pos, _CH - 1)
                    plsc.store_scatter(slist, [chi, off], s_st[sl], mask=msk)
                    plsc.store_scatter(dlist, [chi, off], d - win_lo, mask=msk)
                    plsc.store_scatter(tlist, [chi, off], t_st[sl], mask=msk)
                    return jnp.max(pos) + 1
                return lax.fori_loop(0, pre // _LANES, vec_step, cnt)
            cnt = lax.fori_loop(0, n_pre, prep_stage, jnp.int32(0))

            plsc.subcore_barrier()

            n_act = lax.shift_right_arithmetic(cnt + (_CH - 1), 7)
            bufs = (msgbuf, msgbuf2)
            gsems = (gsa, gsb)
            ssems = (ssa, ssb)

            def g_start(ch, b):
                pltpu.async_copy(h_hbm.at[slist.at[ch]], bufs[b], gsems[b])

            def g_wait(b):
                pltpu.make_async_copy(h_hbm.at[slist.at[0]], bufs[b],
                                      gsems[b]).wait()

            def s_start(ch, b):
                pltpu.async_copy(bufs[b], acc.at[dlist.at[ch]], ssems[b],
                                 add=True)

            def s_wait(b):
                pltpu.make_async_copy(bufs[b], acc.at[dlist.at[0]],
                                      ssems[b]).wait()

            def relu_pass(ch, b):
                buf = bufs[b]

                def relu_group(g, carry2):
                    et_vec = tlist[ch, pl.ds(g * _LANES, _LANES)]
                    for rr in range(_LANES):
                        r = g * _LANES + rr
                        et = et_vec[rr]
                        live = (ch * _CH + r) < cnt
                        mf = jnp.where(live, 1.0, 0.0)
                        for j in range(8):
                            sl = pl.ds(j * _LANES, _LANES)
                            v = buf[r, sl] + bond_v[et, sl]
                            buf[r, sl] = jnp.maximum(v, 0.0) * mf
                    return carry2
                lax.fori_loop(0, _CH // _LANES, relu_group, 0)

            @pl.when(n_act > 0)
            def _():
                g_start(0, 0)
            @pl.when(n_act > 1)
            def _():
                g_start(1, 1)

            def pair_body(p, carry):
                for lane in range(2):
                    ch = p * 2 + lane

                    @pl.when(ch < n_act)
                    def _():
                        g_wait(lane)
                        relu_pass(ch, lane)
                        s_start(ch, lane)

                        @pl.when(ch + 2 < n_act)
                        def _():
                            s_wait(lane)
                            g_start(ch + 2, lane)
                return carry
            n_pairs = lax.shift_right_arithmetic(n_act + 1, 1)
            lax.fori_loop(0, n_pairs, pair_body, 0)

            @pl.when(n_act > 0)
            def _():
                s_wait(0)
            @pl.when(n_act > 1)
            def _():
                s_wait(1)

            plsc.subcore_barrier()

            pltpu.sync_copy(acc.at[pl.ds(s * rpt, rpt)],
                            out_hbm.at[pl.ds(win_lo + s * rpt, rpt)])
            plsc.subcore_barrier()
            return wcarry

        lax.fori_loop(0, n_win, win_body, 0)

    return k


_edge_agg_intra = _make_edge_agg(262144, _SK, 4096, 8)
_edge_agg_global = _make_edge_agg(16384, _N_TOTAL, 512, 1)


def _mlp_ln_body(hl_ref, hb_ref, hprev_ref, w1a_ref, w1b_ref, b1_ref,
                 w2_ref, b2_ref, lng_ref, lnb_ref, out_ref):
    hl = hl_ref[...]
    hb = hb_ref[...]
    z = jnp.dot(hl, w1a_ref[...], preferred_element_type=jnp.float32)
    z += jnp.dot(hb, w1b_ref[...], preferred_element_type=jnp.float32)
    z += b1_ref[...]
    z = jax.nn.gelu(z)
    z = jnp.dot(z, w2_ref[...], preferred_element_type=jnp.float32) + b2_ref[...]
    mu = jnp.mean(z, axis=-1, keepdims=True)
    var = jnp.mean((z - mu) ** 2, axis=-1, keepdims=True)
    z = (z - mu) * jax.lax.rsqrt(var + 1e-5) * lng_ref[...] + lnb_ref[...]
    out_ref[...] = hprev_ref[...] + z


def _fused_mlp_ln(h_local, h_bcast, h_prev, w1a, w1b, b1, w2, b2, lng, lnb):
    n = h_local.shape[0]
    grid = (n // _ROW_BLOCK,)
    row_spec = pl.BlockSpec((_ROW_BLOCK, _H), lambda i: (i, 0))
    full = pl.BlockSpec((_H, _H), lambda i: (0, 0))
    vec = pl.BlockSpec((_H,), lambda i: (0,))
    return pl.pallas_call(
        _mlp_ln_body,
        grid=grid,
        in_specs=[row_spec, row_spec, row_spec, full, full, vec, full, vec,
                  vec, vec],
        out_specs=row_spec,
        out_shape=jax.ShapeDtypeStruct((n, _H), jnp.float32),
    )(h_local, h_bcast, h_prev, w1a, w1b, b1, w2, b2, lng, lnb)


def _bn(x, g, b):
    mu = x.mean(0)
    var = x.var(0)
    return (x - mu) / jnp.sqrt(var + 1e-5) * g + b


def _rwse(global_ei, n, t):
    A = jnp.zeros((n, n), dtype=jnp.float32).at[global_ei[0], global_ei[1]].add(1.0)
    deg = jnp.clip(A.sum(1), 1.0, None)
    P = A / deg[:, None]
    half = t // 2
    pows = [P]
    for _ in range(half - 1):
        pows.append(pows[-1] @ P)
    diags = []
    for step in range(1, t + 1):
        a = (step + 1) // 2
        b = step // 2
        if b == 0:
            diags.append(jnp.diagonal(pows[a - 1]))
        else:
            diags.append(jnp.sum(pows[a - 1] * pows[b - 1].T, axis=1))
    return jnp.stack(diags, axis=1)


def kernel(params, x_idx, intra_ei, intra_ea_idx, global_ei, global_ea_idx,
           node_ids, batch_assign, valid):
    bond = params["bond_emb"]
    rwse = _rwse(global_ei, _N_TOTAL, _T_RWSE)
    h0 = params["atom_emb"][x_idx]
    rwse_flat = jax.nn.relu(rwse @ params["rwse_W"] + params["rwse_b"])[node_ids]
    valid_f = valid.astype(jnp.float32)[:, None]
    h = (h0 + rwse_flat) * valid_f
    isrc = intra_ei[0]
    idst = intra_ei[1]
    gsrc = global_ei[0]
    gdst = global_ei[1]
    for lp in params["layers"]:
        agg = _edge_agg_intra(h, isrc, idst, intra_ea_idx, bond)
        z = (1.0 + lp["eps_l"]) * h + agg
        u = jax.nn.relu(jax.nn.relu(z @ lp["W1_l"] + lp["b1_l"]) @ lp["W2_l"]
                        + lp["b2_l"])
        h_local = _bn(u, lp["bn_g_l"], lp["bn_b_l"])

        h_node = h.reshape(_N_TOTAL, _M_SUB * _K, _H)[:, ::_K].mean(axis=1)
        agg_g = _edge_agg_global(h_node, gsrc, gdst, global_ea_idx, bond)
        zg = (1.0 + lp["eps_g"]) * h_node + agg_g
        ug = jax.nn.relu(jax.nn.relu(zg @ lp["W1_g"] + lp["b1_g"]) @ lp["W2_g"]
                         + lp["b2_g"])
        h_node = h_node + _bn(ug, lp["bn_g_g"], lp["bn_b_g"])

        hb_node = h_node @ lp["W_bcast"]
        h_bcast = jnp.repeat(hb_node, _M_SUB * _K, axis=0)

        w1a = lp["cW1"][:_H]
        w1b = lp["cW1"][_H:]
        h_new = _fused_mlp_ln(h_local, h_bcast, h, w1a, w1b, lp["cb1"],
                              lp["cW2"], lp["cb2"], lp["ln_g"], lp["ln_b"])
        h = h_new * valid_f
    hv = h * valid_f
    cnt = valid_f[:, 0].reshape(_S, _K).sum(axis=1)
    h_sub = hv.reshape(_S, _K, _H).sum(axis=1) / jnp.clip(cnt, 1.0, None)[:, None]
    h_node_f = h_sub.reshape(_N_TOTAL, _M_SUB, _H).mean(axis=1)
    onehot = (batch_assign[None, :] == jnp.arange(_N_GRAPHS)[:, None]
              ).astype(jnp.float32)
    return onehot @ h_node_f


# speedup vs baseline: 1.4876x; 1.0342x over previous
import functools
import jax
import jax.numpy as jnp
from jax import lax
from jax.experimental import pallas as pl
from jax.experimental.pallas import tpu as pltpu
from jax.experimental.pallas import tpu_sc as plsc

_N_TOTAL = 1024
_M_SUB = 2
_K = 32
_S = _N_TOTAL * _M_SUB
_SK = _S * _K
_H = 128
_T_RWSE = 16
_N_GRAPHS = 64
_EDGE_DIM = 16

_ROW_BLOCK = 1024
_CH = 128
_LANES = 16
_N_TILES = 16
_N_CORES = 2


def _make_edge_agg(n_edges, n_rows, win_rows, n_win):
    epc = n_edges // _N_TILES
    n_chunks = epc // _CH
    pre = min(epc, 2048)
    n_pre = epc // pre
    rpt = win_rows // _N_TILES
    mesh = plsc.VectorSubcoreMesh(core_axis_name="c", subcore_axis_name="s",
                                  num_cores=_N_CORES, num_subcores=_N_TILES)

    @functools.partial(
        pl.kernel, mesh=mesh,
        compiler_params=pltpu.CompilerParams(needs_layout_passes=False),
        out_type=jax.ShapeDtypeStruct((n_rows, _H), jnp.float32),
        scratch_types=[
            pltpu.VMEM((_CH, _H), jnp.float32),
            pltpu.VMEM((_CH, _H), jnp.float32),
            pltpu.VMEM((n_chunks, _CH), jnp.int32),
            pltpu.VMEM((n_chunks, _CH), jnp.int32),
            pltpu.VMEM((n_chunks, _CH), jnp.int32),
            pltpu.VMEM((pre,), jnp.int32),
            pltpu.VMEM((pre,), jnp.int32),
            pltpu.VMEM((pre,), jnp.int32),
            pltpu.VMEM((_EDGE_DIM, _H), jnp.float32),
            pltpu.VMEM_SHARED((win_rows + 8, _H), jnp.float32),
            pltpu.SemaphoreType.DMA,
            pltpu.SemaphoreType.DMA,
            pltpu.SemaphoreType.DMA,
            pltpu.SemaphoreType.DMA,
        ],
    )
    def k(h_hbm, src_hbm, dst_hbm, et_hbm, bond_hbm, out_hbm,
          msgbuf, msgbuf2, slist, dlist, tlist, s_st, d_st, t_st, bond_v, acc,
          gsa, gsb, ssa, ssb):
        c = lax.axis_index("c")
        s = lax.axis_index("s")
        ebase = s * epc
        zf = jnp.zeros((_LANES,), jnp.float32)
        zi = jnp.zeros((_LANES,), jnp.int32)
        pltpu.sync_copy(bond_hbm, bond_v)

        def win_body(w, wcarry):
            win_lo = (c * n_win + w) * win_rows

            def zrow(r, carry):
                for j in range(8):
                    msgbuf[r, pl.ds(j * _LANES, _LANES)] = zf
                return carry
            lax.fori_loop(0, _CH, zrow, 0)

            for q in range(max(1, rpt // _CH)):
                rows = min(rpt, _CH)
                pltpu.sync_copy(msgbuf.at[pl.ds(0, rows)],
                                acc.at[pl.ds(s * rpt + q * _CH, rows)])

            dump = jnp.full((_LANES,), win_rows, jnp.int32)

            def zlist(i, carry):
                for j in range(8):
                    sl = pl.ds(j * _LANES, _LANES)
                    slist[i, sl] = zi
                    dlist[i, sl] = dump
                    tlist[i, sl] = zi
                return carry
            lax.fori_loop(0, n_chunks, zlist, 0)

            def prep_stage(p, cnt):
                start = ebase + p * pre
                pltpu.sync_copy(src_hbm.at[pl.ds(start, pre)], s_st)
                pltpu.sync_copy(dst_hbm.at[pl.ds(start, pre)], d_st)
                pltpu.sync_copy(et_hbm.at[pl.ds(start, pre)], t_st)

                def vec_step(i, cnt):
                    sl = pl.ds(i * _LANES, _LANES)
                    d = d_st[sl]
                    msk = (d >= win_lo) & (d < win_lo + win_rows)
                    pos = cnt + plsc.cumsum(jnp.where(msk, 1, 0)) - 1
                    chi = lax.shift_right_arithmetic(pos, 7)
                    off = lax.bitwise_and(pos, _CH - 1)
                    plsc.store_scatter(slist, [chi, off], s_st[sl], mask=msk)
                    plsc.store_scatter(dlist, [chi, off], d - win_lo, mask=msk)
                    plsc.store_scatter(tlist, [chi, off], t_st[sl], mask=msk)
                    return cnt + plsc.all_reduce_population_count(msk)[0]
                return lax.fori_loop(0, pre // _LANES, vec_step, cnt)
            cnt = lax.fori_loop(0, n_pre, prep_stage, jnp.int32(0))

            plsc.subcore_barrier()

            n_act = lax.shift_right_arithmetic(cnt + (_CH - 1), 7)
            bufs = (msgbuf, msgbuf2)
            gsems = (gsa, gsb)
            ssems = (ssa, ssb)

            def g_start(ch, b):
                pltpu.async_copy(h_hbm.at[slist.at[ch]], bufs[b], gsems[b])

            def g_wait(b):
                pltpu.make_async_copy(h_hbm.at[slist.at[0]], bufs[b],
                                      gsems[b]).wait()

            def s_start(ch, b):
                pltpu.async_copy(bufs[b], acc.at[dlist.at[ch]], ssems[b],
                                 add=True)

            def s_wait(b):
                pltpu.make_async_copy(bufs[b], acc.at[dlist.at[0]],
                                      ssems[b]).wait()

            def relu_pass(ch, b):
                buf = bufs[b]

                def relu_group(g, carry2):
                    et_vec = tlist[ch, pl.ds(g * _LANES, _LANES)]
                    for rr in range(_LANES):
                        r = g * _LANES + rr
                        et = et_vec[rr]
                        for j in range(8):
                            sl = pl.ds(j * _LANES, _LANES)
                            v = buf[r, sl] + bond_v[et, sl]
                            buf[r, sl] = jnp.maximum(v, 0.0)
                    return carry2
                lax.fori_loop(0, _CH // _LANES, relu_group, 0)

            @pl.when(n_act > 0)
            def _():
                g_start(0, 0)
            @pl.when(n_act > 1)
            def _():
                g_start(1, 1)

            def pair_body(p, carry):
                for lane in range(2):
                    ch = p * 2 + lane

                    @pl.when(ch < n_act)
                    def _():
                        g_wait(lane)
                        relu_pass(ch, lane)
                        s_start(ch, lane)

                        @pl.when(ch + 2 < n_act)
                        def _():
                            s_wait(lane)
                            g_start(ch + 2, lane)
                return carry
            n_pairs = lax.shift_right_arithmetic(n_act + 1, 1)
            lax.fori_loop(0, n_pairs, pair_body, 0)

            @pl.when(n_act > 0)
            def _():
                s_wait(0)
            @pl.when(n_act > 1)
            def _():
                s_wait(1)

            plsc.subcore_barrier()

            pltpu.sync_copy(acc.at[pl.ds(s * rpt, rpt)],
                            out_hbm.at[pl.ds(win_lo + s * rpt, rpt)])
            plsc.subcore_barrier()
            return wcarry

        lax.fori_loop(0, n_win, win_body, 0)

    return k


_edge_agg_intra = _make_edge_agg(262144, _SK, 4096, 8)
_edge_agg_global = _make_edge_agg(16384, _N_TOTAL, 512, 1)


def _mlp_ln_body(hl_ref, hb_ref, hprev_ref, w1a_ref, w1b_ref, b1_ref,
                 w2_ref, b2_ref, lng_ref, lnb_ref, out_ref):
    hl = hl_ref[...]
    hb = hb_ref[...]
    z = jnp.dot(hl, w1a_ref[...], preferred_element_type=jnp.float32)
    z += jnp.dot(hb, w1b_ref[...], preferred_element_type=jnp.float32)
    z += b1_ref[...]
    z = jax.nn.gelu(z)
    z = jnp.dot(z, w2_ref[...], preferred_element_type=jnp.float32) + b2_ref[...]
    mu = jnp.mean(z, axis=-1, keepdims=True)
    var = jnp.mean((z - mu) ** 2, axis=-1, keepdims=True)
    z = (z - mu) * jax.lax.rsqrt(var + 1e-5) * lng_ref[...] + lnb_ref[...]
    out_ref[...] = hprev_ref[...] + z


def _fused_mlp_ln(h_local, h_bcast, h_prev, w1a, w1b, b1, w2, b2, lng, lnb):
    n = h_local.shape[0]
    grid = (n // _ROW_BLOCK,)
    row_spec = pl.BlockSpec((_ROW_BLOCK, _H), lambda i: (i, 0))
    full = pl.BlockSpec((_H, _H), lambda i: (0, 0))
    vec = pl.BlockSpec((_H,), lambda i: (0,))
    return pl.pallas_call(
        _mlp_ln_body,
        grid=grid,
        in_specs=[row_spec, row_spec, row_spec, full, full, vec, full, vec,
                  vec, vec],
        out_specs=row_spec,
        out_shape=jax.ShapeDtypeStruct((n, _H), jnp.float32),
    )(h_local, h_bcast, h_prev, w1a, w1b, b1, w2, b2, lng, lnb)


def _bn(x, g, b):
    mu = x.mean(0)
    var = x.var(0)
    return (x - mu) / jnp.sqrt(var + 1e-5) * g + b


def _rwse(global_ei, n, t):
    A = jnp.zeros((n, n), dtype=jnp.float32).at[global_ei[0], global_ei[1]].add(1.0)
    deg = jnp.clip(A.sum(1), 1.0, None)
    P = A / deg[:, None]
    half = t // 2
    pows = [P]
    for _ in range(half - 1):
        pows.append(pows[-1] @ P)
    diags = []
    for step in range(1, t + 1):
        a = (step + 1) // 2
        b = step // 2
        if b == 0:
            diags.append(jnp.diagonal(pows[a - 1]))
        else:
            diags.append(jnp.sum(pows[a - 1] * pows[b - 1].T, axis=1))
    return jnp.stack(diags, axis=1)


def kernel(params, x_idx, intra_ei, intra_ea_idx, global_ei, global_ea_idx,
           node_ids, batch_assign, valid):
    bond = params["bond_emb"]
    rwse = _rwse(global_ei, _N_TOTAL, _T_RWSE)
    h0 = params["atom_emb"][x_idx]
    rwse_flat = jax.nn.relu(rwse @ params["rwse_W"] + params["rwse_b"])[node_ids]
    valid_f = valid.astype(jnp.float32)[:, None]
    h = (h0 + rwse_flat) * valid_f
    isrc = intra_ei[0]
    idst = intra_ei[1]
    gsrc = global_ei[0]
    gdst = global_ei[1]
    for lp in params["layers"]:
        agg = _edge_agg_intra(h, isrc, idst, intra_ea_idx, bond)
        z = (1.0 + lp["eps_l"]) * h + agg
        u = jax.nn.relu(jax.nn.relu(z @ lp["W1_l"] + lp["b1_l"]) @ lp["W2_l"]
                        + lp["b2_l"])
        h_local = _bn(u, lp["bn_g_l"], lp["bn_b_l"])

        h_node = h.reshape(_N_TOTAL, _M_SUB * _K, _H)[:, ::_K].mean(axis=1)
        agg_g = _edge_agg_global(h_node, gsrc, gdst, global_ea_idx, bond)
        zg = (1.0 + lp["eps_g"]) * h_node + agg_g
        ug = jax.nn.relu(jax.nn.relu(zg @ lp["W1_g"] + lp["b1_g"]) @ lp["W2_g"]
                         + lp["b2_g"])
        h_node = h_node + _bn(ug, lp["bn_g_g"], lp["bn_b_g"])

        hb_node = h_node @ lp["W_bcast"]
        h_bcast = jnp.repeat(hb_node, _M_SUB * _K, axis=0)

        w1a = lp["cW1"][:_H]
        w1b = lp["cW1"][_H:]
        h_new = _fused_mlp_ln(h_local, h_bcast, h, w1a, w1b, lp["cb1"],
                              lp["cW2"], lp["cb2"], lp["ln_g"], lp["ln_b"])
        h = h_new * valid_f
    hv = h * valid_f
    cnt = valid_f[:, 0].reshape(_S, _K).sum(axis=1)
    h_sub = hv.reshape(_S, _K, _H).sum(axis=1) / jnp.clip(cnt, 1.0, None)[:, None]
    h_node_f = h_sub.reshape(_N_TOTAL, _M_SUB, _H).mean(axis=1)
    onehot = (batch_assign[None, :] == jnp.arange(_N_GRAPHS)[:, None]
              ).astype(jnp.float32)
    return onehot @ h_node_f
